# Initial kernel scaffold; baseline (speedup 1.0000x reference)
import jax
import jax.numpy as jnp
from jax.experimental import pallas as pl


def kernel(x, adj_indices, adj_values, W1, b1, W2, b2, Wc1, bc1, Wc2, bc2, Wc3, bc3, Wc4, bc4):
    raise NotImplementedError("write your pallas kernel here")


# speedup vs baseline: 3.4263x; 3.4263x over previous
import dataclasses
import functools

import jax
import jax.numpy as jnp
from jax import lax
from jax.experimental import pallas as pl
from jax.experimental.pallas import tpu as pltpu
from jax.experimental.pallas import tpu_sc as plsc

_CORES = 2
_SUBCORES = 16
_NW = _CORES * _SUBCORES
_CHUNK = 128
_BR = 1000


def _spmm_partials(mat, src, dst, vals, n_chunks):
    NP, D = mat.shape
    nvec = D // 16
    rows_per_tile = NP // _SUBCORES
    epw = n_chunks * _CHUNK
    mesh = plsc.VectorSubcoreMesh(core_axis_name="c", subcore_axis_name="s")
    cp = pltpu.CompilerParams()
    if "needs_layout_passes" in pltpu.CompilerParams.__dataclass_fields__:
        cp = dataclasses.replace(cp, needs_layout_passes=False)

    @functools.partial(
        pl.kernel,
        compiler_params=cp,
        out_type=jax.ShapeDtypeStruct((_CORES, NP, D), jnp.float32),
        mesh=mesh,
        scratch_types=[
            pltpu.VMEM((_CHUNK,), jnp.int32),
            pltpu.VMEM((_CHUNK,), jnp.int32),
            pltpu.VMEM((_CHUNK,), jnp.float32),
            pltpu.VMEM((_CHUNK, D), jnp.float32),
            pltpu.VMEM_SHARED((NP, D), jnp.float32),
            pltpu.SemaphoreType.DMA,
        ],
    )
    def k(mat_hbm, src_hbm, dst_hbm, vals_hbm, out_hbm,
          src_v, dst_v, vals_v, rows_v, acc, sem):
        cid = lax.axis_index("c")
        sid = lax.axis_index("s")
        wid = sid * _CORES + cid

        zeros = jnp.zeros((16,), jnp.float32)

        @pl.loop(0, _CHUNK)
        def _(r):
            for v in range(nvec):
                rows_v[r, pl.ds(v * 16, 16)] = zeros

        base = sid * rows_per_tile
        for i in range(rows_per_tile // _CHUNK):
            pltpu.sync_copy(rows_v, acc.at[pl.ds(base + i * _CHUNK, _CHUNK)])
        plsc.subcore_barrier()

        ebase = wid * epw

        @pl.loop(0, n_chunks)
        def _(ci):
            off = ebase + ci * _CHUNK
            pltpu.sync_copy(src_hbm.at[pl.ds(off, _CHUNK)], src_v)
            pltpu.sync_copy(dst_hbm.at[pl.ds(off, _CHUNK)], dst_v)
            pltpu.sync_copy(vals_hbm.at[pl.ds(off, _CHUNK)], vals_v)
            pltpu.async_copy(mat_hbm.at[src_v], rows_v, sem).wait()

            @pl.loop(0, _CHUNK)
            def _(e):
                val = plsc.load_gather(vals_v, [jnp.full((16,), e, jnp.int32)])
                for v in range(nvec):
                    sl = pl.ds(v * 16, 16)
                    rows_v[e, sl] = rows_v[e, sl] * val

            pltpu.sync_copy(rows_v, acc.at[dst_v], add=True)

        plsc.subcore_barrier()
        pltpu.sync_copy(acc.at[pl.ds(base, rows_per_tile)],
                        out_hbm.at[cid, pl.ds(base, rows_per_tile)])

    return k(mat, src, dst, vals)


def _matmul(x, W, out_rows):
    N, K = x.shape
    F = W.shape[1]

    def body(x_ref, w_ref, o_ref):
        o_ref[...] = jnp.dot(x_ref[...], w_ref[...],
                             preferred_element_type=jnp.float32)

    return pl.pallas_call(
        body,
        grid=(N // _BR,),
        in_specs=[pl.BlockSpec((_BR, K), lambda i: (i, 0)),
                  pl.BlockSpec((K, F), lambda i: (0, 0))],
        out_specs=pl.BlockSpec((_BR, F), lambda i: (i, 0)),
        out_shape=jax.ShapeDtypeStruct((out_rows, F), jnp.float32),
    )(x, W)


def _mid(P, b, W, n_rows):
    NP = P.shape[1]
    K = P.shape[2]
    F = W.shape[1]

    def body(p0_ref, p1_ref, b_ref, w_ref, o_ref):
        h = jax.nn.relu(p0_ref[0] + p1_ref[0] + b_ref[...])
        o_ref[...] = jnp.dot(h, w_ref[...], preferred_element_type=jnp.float32)

    return pl.pallas_call(
        body,
        grid=(n_rows // _BR,),
        in_specs=[pl.BlockSpec((1, _BR, K), lambda i: (0, i, 0)),
                  pl.BlockSpec((1, _BR, K), lambda i: (1, i, 0)),
                  pl.BlockSpec((1, K), lambda i: (0, 0)),
                  pl.BlockSpec((K, F), lambda i: (0, 0))],
        out_specs=pl.BlockSpec((_BR, F), lambda i: (i, 0)),
        out_shape=jax.ShapeDtypeStruct((NP, F), jnp.float32),
    )(P, P, b, W)


def _head(Q, b2, Wc1, bc1, Wc2, bc2, Wc3, bc3, Wc4, bc4, n_rows):
    K = Q.shape[2]
    C = Wc4.shape[1]

    def body(p0_ref, p1_ref, b2_ref, w1_ref, b1_ref, w2_ref, bb2_ref,
             w3_ref, b3_ref, w4_ref, b4_ref, o_ref):
        h = p0_ref[0] + p1_ref[0] + b2_ref[...]
        z = jax.nn.relu(jnp.dot(h, w1_ref[...],
                                preferred_element_type=jnp.float32) + b1_ref[...])
        z = jax.nn.relu(jnp.dot(z, w2_ref[...],
                                preferred_element_type=jnp.float32) + bb2_ref[...])
        z = jax.nn.relu(jnp.dot(z, w3_ref[...],
                                preferred_element_type=jnp.float32) + b3_ref[...])
        logits = jnp.dot(z, w4_ref[...],
                         preferred_element_type=jnp.float32) + b4_ref[...]
        m = jnp.max(logits, axis=1, keepdims=True)
        e = jnp.exp(logits - m)
        o_ref[...] = e / jnp.sum(e, axis=1, keepdims=True)

    def full(shape):
        return pl.BlockSpec(shape, lambda i: tuple(0 for _ in shape))

    return pl.pallas_call(
        body,
        grid=(n_rows // _BR,),
        in_specs=[pl.BlockSpec((1, _BR, K), lambda i: (0, i, 0)),
                  pl.BlockSpec((1, _BR, K), lambda i: (1, i, 0)),
                  full((1, K)),
                  full(Wc1.shape), full((1, 2 * K)),
                  full(Wc2.shape), full((1, 2 * K)),
                  full(Wc3.shape), full((1, 2 * K)),
                  full(Wc4.shape), full((1, C))],
        out_specs=pl.BlockSpec((_BR, C), lambda i: (i, 0)),
        out_shape=jax.ShapeDtypeStruct((n_rows, C), jnp.float32),
    )(Q, Q, b2, Wc1, bc1, Wc2, bc2, Wc3, bc3, Wc4, bc4)


def kernel(x, adj_indices, adj_values, W1, b1, W2, b2,
           Wc1, bc1, Wc2, bc2, Wc3, bc3, Wc4, bc4):
    N = x.shape[0]
    NP = -(-N // (_SUBCORES * _CHUNK)) * (_SUBCORES * _CHUNK)
    E = adj_values.shape[0]
    n_chunks = -(-E // (_NW * _CHUNK))
    E_pad = _NW * n_chunks * _CHUNK

    dst = adj_indices[0]
    src = adj_indices[1]
    pad = E_pad - E
    if pad:
        zi = jnp.zeros((pad,), jnp.int32)
        dst = jnp.concatenate([dst, zi])
        src = jnp.concatenate([src, zi])
        vals = jnp.concatenate([adj_values, jnp.zeros((pad,), jnp.float32)])
    else:
        vals = adj_values

    M1 = _matmul(x, W1, NP)
    P = _spmm_partials(M1, src, dst, vals, n_chunks)
    M2 = _mid(P, b1[None, :], W2, N)
    Q = _spmm_partials(M2, src, dst, vals, n_chunks)
    return _head(Q, b2[None, :],
                 Wc1, bc1[None, :], Wc2, bc2[None, :],
                 Wc3, bc3[None, :], Wc4, bc4[None, :], N)


# speedup vs baseline: 3.5697x; 1.0418x over previous
import dataclasses
import functools

import jax
import jax.numpy as jnp
from jax import lax
from jax.experimental import pallas as pl
from jax.experimental.pallas import tpu as pltpu
from jax.experimental.pallas import tpu_sc as plsc

_CORES = 2
_SUBCORES = 16
_NW = _CORES * _SUBCORES
_CHUNK = 128
_BR = 1000


def _spmm_partials(mat, src, dst, vals, n_chunks):
    NP, D = mat.shape
    nvec = D // 16
    rows_per_tile = NP // _SUBCORES
    mesh = plsc.VectorSubcoreMesh(core_axis_name="c", subcore_axis_name="s")
    cp = pltpu.CompilerParams()
    if "needs_layout_passes" in pltpu.CompilerParams.__dataclass_fields__:
        cp = dataclasses.replace(cp, needs_layout_passes=False)

    @functools.partial(
        pl.kernel,
        compiler_params=cp,
        out_type=jax.ShapeDtypeStruct((_CORES, NP, D), jnp.float32),
        mesh=mesh,
        scratch_types=[
            pltpu.VMEM((n_chunks, _CHUNK), jnp.int32),
            pltpu.VMEM((_CHUNK,), jnp.int32),
            pltpu.VMEM((_CHUNK,), jnp.int32),
            pltpu.VMEM((_CHUNK,), jnp.float32),
            pltpu.VMEM((_CHUNK,), jnp.float32),
            pltpu.VMEM((_CHUNK, D), jnp.float32),
            pltpu.VMEM((_CHUNK, D), jnp.float32),
            pltpu.VMEM_SHARED((NP, D), jnp.float32),
            pltpu.SemaphoreType.DMA,
            pltpu.SemaphoreType.DMA,
            pltpu.SemaphoreType.DMA,
            pltpu.SemaphoreType.DMA,
        ],
    )
    def k(mat_hbm, src_hbm, dst_hbm, vals_hbm, out_hbm,
          src_all, dst_a, dst_b, vals_a, vals_b, rows_a, rows_b, acc,
          sem_a, sem_b, sem_ia, sem_ib):
        cid = lax.axis_index("c")
        sid = lax.axis_index("s")
        wid = sid * _CORES + cid

        pltpu.sync_copy(src_hbm.at[wid], src_all)

        def prefetch(ci, rows, dst_v, vals_v, sem, sem_i):
            ck = lax.rem(ci, n_chunks)
            off = ck * _CHUNK
            pltpu.async_copy(mat_hbm.at[src_all.at[ck]], rows, sem)
            pltpu.async_copy(dst_hbm.at[wid, pl.ds(off, _CHUNK)], dst_v, sem_i)
            pltpu.async_copy(vals_hbm.at[wid, pl.ds(off, _CHUNK)], vals_v, sem_i)

        def wait(rows, dst_v, vals_v, sem, sem_i):
            pltpu.make_async_copy(mat_hbm.at[src_all.at[0]], rows, sem).wait()
            pltpu.make_async_copy(dst_hbm.at[0, pl.ds(0, _CHUNK)], dst_v,
                                  sem_i).wait()
            pltpu.make_async_copy(vals_hbm.at[0, pl.ds(0, _CHUNK)], vals_v,
                                  sem_i).wait()

        prefetch(0, rows_a, dst_a, vals_a, sem_a, sem_ia)

        zeros = jnp.zeros((16,), jnp.float32)

        @pl.loop(0, _CHUNK)
        def _(r):
            for v in range(nvec):
                rows_b[r, pl.ds(v * 16, 16)] = zeros

        base = sid * rows_per_tile
        for i in range(rows_per_tile // _CHUNK):
            pltpu.sync_copy(rows_b, acc.at[pl.ds(base + i * _CHUNK, _CHUNK)])
        plsc.subcore_barrier()

        def scale(rows, vals_v):
            @pl.loop(0, _CHUNK, step=4)
            def _(eb):
                for kk in range(4):
                    e = eb + kk
                    val = plsc.load_gather(
                        vals_v, [jnp.full((16,), e, jnp.int32)])
                    for v in range(nvec):
                        sl = pl.ds(v * 16, 16)
                        rows[e, sl] = rows[e, sl] * val

        @pl.loop(0, n_chunks, step=2)
        def _(ci):
            prefetch(ci + 1, rows_b, dst_b, vals_b, sem_b, sem_ib)
            wait(rows_a, dst_a, vals_a, sem_a, sem_ia)
            scale(rows_a, vals_a)
            pltpu.sync_copy(rows_a, acc.at[dst_a], add=True)
            prefetch(ci + 2, rows_a, dst_a, vals_a, sem_a, sem_ia)
            wait(rows_b, dst_b, vals_b, sem_b, sem_ib)
            scale(rows_b, vals_b)
            pltpu.sync_copy(rows_b, acc.at[dst_b], add=True)

        wait(rows_a, dst_a, vals_a, sem_a, sem_ia)

        plsc.subcore_barrier()
        pltpu.sync_copy(acc.at[pl.ds(base, rows_per_tile)],
                        out_hbm.at[cid, pl.ds(base, rows_per_tile)])

    return k(mat, src, dst, vals)


def _matmul(x, W, out_rows):
    N, K = x.shape
    F = W.shape[1]

    def body(x_ref, w_ref, o_ref):
        o_ref[...] = jnp.dot(x_ref[...], w_ref[...],
                             preferred_element_type=jnp.float32)

    return pl.pallas_call(
        body,
        grid=(N // _BR,),
        in_specs=[pl.BlockSpec((_BR, K), lambda i: (i, 0)),
                  pl.BlockSpec((K, F), lambda i: (0, 0))],
        out_specs=pl.BlockSpec((_BR, F), lambda i: (i, 0)),
        out_shape=jax.ShapeDtypeStruct((out_rows, F), jnp.float32),
    )(x, W)


def _mid(P, b, W, n_rows):
    NP = P.shape[1]
    K = P.shape[2]
    F = W.shape[1]

    def body(p0_ref, p1_ref, b_ref, w_ref, o_ref):
        h = jax.nn.relu(p0_ref[0] + p1_ref[0] + b_ref[...])
        o_ref[...] = jnp.dot(h, w_ref[...], preferred_element_type=jnp.float32)

    return pl.pallas_call(
        body,
        grid=(n_rows // _BR,),
        in_specs=[pl.BlockSpec((1, _BR, K), lambda i: (0, i, 0)),
                  pl.BlockSpec((1, _BR, K), lambda i: (1, i, 0)),
                  pl.BlockSpec((1, K), lambda i: (0, 0)),
                  pl.BlockSpec((K, F), lambda i: (0, 0))],
        out_specs=pl.BlockSpec((_BR, F), lambda i: (i, 0)),
        out_shape=jax.ShapeDtypeStruct((NP, F), jnp.float32),
    )(P, P, b, W)


def _head(Q, b2, Wc1, bc1, Wc2, bc2, Wc3, bc3, Wc4, bc4, n_rows):
    K = Q.shape[2]
    C = Wc4.shape[1]

    def body(p0_ref, p1_ref, b2_ref, w1_ref, b1_ref, w2_ref, bb2_ref,
             w3_ref, b3_ref, w4_ref, b4_ref, o_ref):
        h = p0_ref[0] + p1_ref[0] + b2_ref[...]
        z = jax.nn.relu(jnp.dot(h, w1_ref[...],
                                preferred_element_type=jnp.float32) + b1_ref[...])
        z = jax.nn.relu(jnp.dot(z, w2_ref[...],
                                preferred_element_type=jnp.float32) + bb2_ref[...])
        z = jax.nn.relu(jnp.dot(z, w3_ref[...],
                                preferred_element_type=jnp.float32) + b3_ref[...])
        logits = jnp.dot(z, w4_ref[...],
                         preferred_element_type=jnp.float32) + b4_ref[...]
        m = jnp.max(logits, axis=1, keepdims=True)
        e = jnp.exp(logits - m)
        o_ref[...] = e / jnp.sum(e, axis=1, keepdims=True)

    def full(shape):
        return pl.BlockSpec(shape, lambda i: tuple(0 for _ in shape))

    return pl.pallas_call(
        body,
        grid=(n_rows // _BR,),
        in_specs=[pl.BlockSpec((1, _BR, K), lambda i: (0, i, 0)),
                  pl.BlockSpec((1, _BR, K), lambda i: (1, i, 0)),
                  full((1, K)),
                  full(Wc1.shape), full((1, 2 * K)),
                  full(Wc2.shape), full((1, 2 * K)),
                  full(Wc3.shape), full((1, 2 * K)),
                  full(Wc4.shape), full((1, C))],
        out_specs=pl.BlockSpec((_BR, C), lambda i: (i, 0)),
        out_shape=jax.ShapeDtypeStruct((n_rows, C), jnp.float32),
    )(Q, Q, b2, Wc1, bc1, Wc2, bc2, Wc3, bc3, Wc4, bc4)


def kernel(x, adj_indices, adj_values, W1, b1, W2, b2,
           Wc1, bc1, Wc2, bc2, Wc3, bc3, Wc4, bc4):
    N = x.shape[0]
    NP = -(-N // (_SUBCORES * _CHUNK)) * (_SUBCORES * _CHUNK)
    E = adj_values.shape[0]
    n_chunks = -(-E // (_NW * _CHUNK))
    n_chunks += n_chunks % 2
    E_pad = _NW * n_chunks * _CHUNK

    dst = adj_indices[0]
    src = adj_indices[1]
    pad = E_pad - E
    if pad:
        zi = jnp.zeros((pad,), jnp.int32)
        dst = jnp.concatenate([dst, zi])
        src = jnp.concatenate([src, zi])
        vals = jnp.concatenate([adj_values, jnp.zeros((pad,), jnp.float32)])
    else:
        vals = adj_values
    src = src.reshape(_NW, n_chunks, _CHUNK)
    dst = dst.reshape(_NW, n_chunks * _CHUNK)
    vals = vals.reshape(_NW, n_chunks * _CHUNK)

    M1 = _matmul(x, W1, NP)
    P = _spmm_partials(M1, src, dst, vals, n_chunks)
    M2 = _mid(P, b1[None, :], W2, N)
    Q = _spmm_partials(M2, src, dst, vals, n_chunks)
    return _head(Q, b2[None, :],
                 Wc1, bc1[None, :], Wc2, bc2[None, :],
                 Wc3, bc3[None, :], Wc4, bc4[None, :], N)
